# Initial kernel scaffold; baseline (speedup 1.0000x reference)
import jax
import jax.numpy as jnp
from jax.experimental import pallas as pl


def kernel(x, gate_w, gate_b, expert_w, expert_b):
    raise NotImplementedError("write your pallas kernel here")


# speedup vs baseline: 2.2565x; 2.2565x over previous
import functools

import jax
import jax.numpy as jnp
from jax.experimental import pallas as pl
from jax.experimental.pallas import tpu as pltpu

NUM_EXPERTS = 8
TOP_K = 2
DIM = 240
BT = 1024


def _moe_block(x_ref, gw_ref, gb_ref, ew_ref, eb_ref, o_ref):
    xb = x_ref[...]

    logits = jax.lax.dot_general(
        xb, gw_ref[...], (((1,), (1,)), ((), ())),
        preferred_element_type=jnp.float32,
    ) + gb_ref[...]

    idx = jax.lax.broadcasted_iota(jnp.int32, (BT, NUM_EXPERTS), 1)
    m1 = jnp.max(logits, axis=1, keepdims=True)
    i1 = jnp.min(jnp.where(logits == m1, idx, NUM_EXPERTS), axis=1, keepdims=True)
    masked = jnp.where(idx == i1, -jnp.inf, logits)
    m2 = jnp.max(masked, axis=1, keepdims=True)
    i2 = jnp.min(jnp.where(masked == m2, idx, NUM_EXPERTS), axis=1, keepdims=True)
    e2 = jnp.exp(m2 - m1)
    denom = 1.0 + e2
    w1 = 1.0 / denom
    w2 = e2 / denom

    acc = jnp.zeros((BT, DIM), jnp.float32)
    for e in range(NUM_EXPERTS):
        we = jnp.where(i1 == e, w1, 0.0) + jnp.where(i2 == e, w2, 0.0)
        ye = jax.lax.dot_general(
            xb, ew_ref[e], (((1,), (1,)), ((), ())),
            preferred_element_type=jnp.float32,
        )
        acc = acc + we * (ye + eb_ref[e][None, :])
    o_ref[...] = acc


@jax.jit
def kernel(x, gate_w, gate_b, expert_w, expert_b):
    b, s, d = x.shape
    n = b * s
    x2 = x.reshape(n, d)
    gb2 = gate_b.reshape(1, NUM_EXPERTS)

    out = pl.pallas_call(
        _moe_block,
        grid=(n // BT,),
        in_specs=[
            pl.BlockSpec((BT, d), lambda i: (i, 0)),
            pl.BlockSpec((NUM_EXPERTS, d), lambda i: (0, 0)),
            pl.BlockSpec((1, NUM_EXPERTS), lambda i: (0, 0)),
            pl.BlockSpec((NUM_EXPERTS, d, d), lambda i: (0, 0, 0)),
            pl.BlockSpec((NUM_EXPERTS, d), lambda i: (0, 0)),
        ],
        out_specs=pl.BlockSpec((BT, d), lambda i: (i, 0)),
        out_shape=jax.ShapeDtypeStruct((n, d), jnp.float32),
        compiler_params=pltpu.CompilerParams(
            dimension_semantics=("arbitrary",),
        ),
    )(x2, gate_w, gb2, expert_w, expert_b)
    return out.reshape(b, s, d)
